# Initial kernel scaffold; baseline (speedup 1.0000x reference)
import jax
import jax.numpy as jnp
from jax.experimental import pallas as pl


def kernel(token_ids, type_token_ids, attention_mask, tok_table, W, b, seg_table, pos_table):
    raise NotImplementedError("write your pallas kernel here")


# speedup vs baseline: 1.4390x; 1.4390x over previous
import functools

import jax
import jax.numpy as jnp
from jax import lax
from jax.experimental import pallas as pl
from jax.experimental.pallas import tpu as pltpu
from jax.experimental.pallas import tpu_sc as plsc

_B = 1024
_L = 200
_FACT = 64
_EMB = 128
_TOKENS = _B * _L
_NC = 2
_NS = 16
_NW = _NC * _NS
_IDX_W = 128
_IDXROWS = _TOKENS // _IDX_W
_IDXROWS_PER_W = _IDXROWS // _NW
_ROWS_PER_W = _TOKENS // _NW
_K = 10
_GROUPS = _IDXROWS_PER_W // _K


@functools.partial(
    pl.kernel,
    out_type=jax.ShapeDtypeStruct((_TOKENS, _FACT), jnp.float32),
    mesh=plsc.VectorSubcoreMesh(
        core_axis_name="c", subcore_axis_name="s",
        num_cores=_NC, num_subcores=_NS),
    scratch_types=[
        pltpu.VMEM((_IDXROWS_PER_W, _IDX_W), jnp.int32),
        pltpu.VMEM((_K * _IDX_W, _FACT), jnp.float32),
        pltpu.SemaphoreType.DMA,
        pltpu.SemaphoreType.DMA,
    ],
    compiler_params=pltpu.CompilerParams(use_tc_tiling_on_sc=False),
)
def _sc_gather(table_hbm, tok_hbm, out_hbm, idx_v, rows_v, gsem, ssem):
    wid = lax.axis_index("c") * _NS + lax.axis_index("s")
    pltpu.sync_copy(tok_hbm.at[wid], idx_v)

    def group(grp, carry):
        j0 = grp * _K
        gh = []
        for bidx in range(_K):
            gh.append(pltpu.async_copy(
                table_hbm.at[idx_v.at[j0 + bidx]],
                rows_v.at[pl.ds(bidx * _IDX_W, _IDX_W)],
                gsem))
        for h in gh:
            h.wait()
        sh = []
        for bidx in range(_K):
            sh.append(pltpu.async_copy(
                rows_v.at[pl.ds(bidx * _IDX_W, _IDX_W)],
                out_hbm.at[pl.ds(wid * _ROWS_PER_W + (j0 + bidx) * _IDX_W,
                                 _IDX_W)],
                ssem))
        for h in sh:
            h.wait()
        return carry

    lax.fori_loop(0, _GROUPS, group, 0)


_BS = 16


def _tc_body(g_ref, t_ref, w_ref, b_ref, seg_ref, pos_ref, o_ref):
    g = g_ref[...]
    mm = jnp.dot(g.reshape(_BS * _L, _FACT), w_ref[...],
                 preferred_element_type=jnp.float32)
    mm = mm.reshape(_BS, _L, _EMB)
    t = t_ref[...].astype(jnp.float32)[:, :, None]
    s0 = seg_ref[0:1, :].reshape(1, 1, _EMB)
    s1 = seg_ref[1:2, :].reshape(1, 1, _EMB)
    pos = pos_ref[...][None, :, :]
    bias = b_ref[...].reshape(1, 1, _EMB)
    o_ref[...] = mm + bias + pos + s0 + t * (s1 - s0)


_tc_project = pl.pallas_call(
    _tc_body,
    grid=(_B // _BS,),
    in_specs=[
        pl.BlockSpec((_BS, _L, _FACT), lambda i: (i, 0, 0)),
        pl.BlockSpec((_BS, _L), lambda i: (i, 0)),
        pl.BlockSpec((_FACT, _EMB), lambda i: (0, 0)),
        pl.BlockSpec((1, _EMB), lambda i: (0, 0)),
        pl.BlockSpec((2, _EMB), lambda i: (0, 0)),
        pl.BlockSpec((_L, _EMB), lambda i: (0, 0)),
    ],
    out_specs=pl.BlockSpec((_BS, _L, _EMB), lambda i: (i, 0, 0)),
    out_shape=jax.ShapeDtypeStruct((_B, _L, _EMB), jnp.float32),
)


def kernel(token_ids, type_token_ids, attention_mask, tok_table, W, b,
           seg_table, pos_table):
    tokens2d = token_ids.reshape(_NW, _IDXROWS_PER_W, _IDX_W)
    gathered = _sc_gather(tok_table, tokens2d)
    g3 = gathered.reshape(_B, _L, _FACT)
    out = _tc_project(g3, type_token_ids, W, b.reshape(1, _EMB),
                      seg_table, pos_table)
    return (out, attention_mask)


# speedup vs baseline: 1.4391x; 1.0001x over previous
import functools

import jax
import jax.numpy as jnp
from jax import lax
from jax.experimental import pallas as pl
from jax.experimental.pallas import tpu as pltpu
from jax.experimental.pallas import tpu_sc as plsc

_B = 1024
_L = 200
_FACT = 64
_EMB = 128
_TOKENS = _B * _L
_NC = 2
_NS = 16
_NW = _NC * _NS
_IDX_W = 128
_IDXROWS = _TOKENS // _IDX_W
_IDXROWS_PER_W = _IDXROWS // _NW
_ROWS_PER_W = _TOKENS // _NW
_K = 5
_GROUPS = _IDXROWS_PER_W // _K
_LANES = 16


@functools.partial(
    pl.kernel,
    out_type=jax.ShapeDtypeStruct((_TOKENS, _EMB), jnp.float32),
    mesh=plsc.VectorSubcoreMesh(
        core_axis_name="c", subcore_axis_name="s",
        num_cores=_NC, num_subcores=_NS),
    scratch_types=[
        pltpu.VMEM((_IDXROWS_PER_W, _IDX_W), jnp.int32),
        pltpu.VMEM((_K * _IDX_W, _EMB), jnp.float32),
        pltpu.SemaphoreType.DMA,
        pltpu.SemaphoreType.DMA,
    ],
)
def _sc_gather(table_hbm, tok_hbm, out_hbm, idx_v, sup_v, gsem, ssem):
    wid = lax.axis_index("c") * _NS + lax.axis_index("s")
    pltpu.sync_copy(tok_hbm.at[wid], idx_v)

    def halve_row(r, carry):
        for k in range(_IDX_W // _LANES):
            sl = pl.ds(k * _LANES, _LANES)
            idx_v[r, sl] = lax.shift_right_logical(idx_v[r, sl], 1)
        return carry

    lax.fori_loop(0, _IDXROWS_PER_W, halve_row, 0)

    def group(grp, carry):
        j0 = grp * _K
        gh = []
        for b in range(_K):
            gh.append(pltpu.async_copy(
                table_hbm.at[idx_v.at[j0 + b]],
                sup_v.at[pl.ds(b * _IDX_W, _IDX_W)],
                gsem))
        for h in gh:
            h.wait()
        sh = []
        for b in range(_K):
            sh.append(pltpu.async_copy(
                sup_v.at[pl.ds(b * _IDX_W, _IDX_W)],
                out_hbm.at[pl.ds(wid * _ROWS_PER_W + (j0 + b) * _IDX_W,
                                 _IDX_W)],
                ssem))
        for h in sh:
            h.wait()
        return carry

    lax.fori_loop(0, _GROUPS, group, 0)


_BS = 16


def _tc_body(sup_ref, tok_ref, typ_ref, w_ref, b_ref, seg_ref, pos_ref,
             o_ref):
    sup = sup_ref[...]
    par = (tok_ref[...] & 1)[:, :, None]
    g = jnp.where(par == 1, sup[:, :, _FACT:], sup[:, :, :_FACT])
    mm = jnp.dot(g.reshape(_BS * _L, _FACT), w_ref[...],
                 preferred_element_type=jnp.float32)
    mm = mm.reshape(_BS, _L, _EMB)
    t = typ_ref[...].astype(jnp.float32)[:, :, None]
    s0 = seg_ref[0:1, :].reshape(1, 1, _EMB)
    s1 = seg_ref[1:2, :].reshape(1, 1, _EMB)
    pos = pos_ref[...][None, :, :]
    bias = b_ref[...].reshape(1, 1, _EMB)
    o_ref[...] = mm + bias + pos + s0 + t * (s1 - s0)


_tc_project = pl.pallas_call(
    _tc_body,
    grid=(_B // _BS,),
    in_specs=[
        pl.BlockSpec((_BS, _L, _EMB), lambda i: (i, 0, 0)),
        pl.BlockSpec((_BS, _L), lambda i: (i, 0)),
        pl.BlockSpec((_BS, _L), lambda i: (i, 0)),
        pl.BlockSpec((_FACT, _EMB), lambda i: (0, 0)),
        pl.BlockSpec((1, _EMB), lambda i: (0, 0)),
        pl.BlockSpec((2, _EMB), lambda i: (0, 0)),
        pl.BlockSpec((_L, _EMB), lambda i: (0, 0)),
    ],
    out_specs=pl.BlockSpec((_BS, _L, _EMB), lambda i: (i, 0, 0)),
    out_shape=jax.ShapeDtypeStruct((_B, _L, _EMB), jnp.float32),
)


def kernel(token_ids, type_token_ids, attention_mask, tok_table, W, b,
           seg_table, pos_table):
    table2 = tok_table.reshape(_FACT * 1000000 // _EMB, _EMB)
    tokens3d = token_ids.reshape(_NW, _IDXROWS_PER_W, _IDX_W)
    packed = _sc_gather(table2, tokens3d)
    p3 = packed.reshape(_B, _L, _EMB)
    out = _tc_project(p3, token_ids, type_token_ids, W, b.reshape(1, _EMB),
                      seg_table, pos_table)
    return (out, attention_mask)
